# baseline (speedup 1.0000x reference)
import functools

import jax
import jax.numpy as jnp
from jax import lax
from jax.experimental import pallas as pl
from jax.experimental.pallas import tpu as pltpu
from jax.experimental.pallas import tpu_sc as plsc

_NC = 2
_NS = 16
_NW = _NC * _NS
_L = 16
_BT = 128
_BP = _BT + 1

_P = 5


def _body(
    h,
    d,
    clip,
    time_hbm,
    table_hbm,
    out_hbm,
    table_v,
    t_t,
    ob0,
    ob1,
    wsem0,
    wsem1,
):
    wid = lax.axis_index("s") * _NC + lax.axis_index("c")
    n_pairs = h * h
    n_chunks = n_pairs // _P
    n2 = n_chunks // 2
    n_g = _BT // _L

    pltpu.sync_copy(table_hbm, table_v)
    pltpu.sync_copy(time_hbm.at[:, pl.ds(wid * _BT, _BT)], t_t)

    lane = lax.iota(jnp.int32, _L)
    ict0 = lane >> 3
    ics0 = lane & 7
    ict1 = ict0 + (_L >> 3)

    def compute(chunk, ob):
        p0 = chunk * _P
        for p_loc in range(_P):
            p = p0 + p_loc
            i = p // h
            q = p - i * h
            obp = ob.at[p_loc]

            @plsc.parallel_loop(0, n_g, unroll=1)
            def grp(g):
                gl = g * _L
                ti = t_t[i, pl.ds(gl, _L)]
                tq = t_t[q, pl.ds(gl, _L)]
                rows16 = jnp.minimum(jnp.abs(ti - tq), clip)
                wb = rows16 * d
                for l in range(_L):
                    base = wb[l]
                    v0 = table_v[pl.ds(base, _L)]
                    v1 = table_v[pl.ds(base + _L, _L)]
                    bl = jnp.full((_L,), gl + l, jnp.int32)
                    plsc.store_scatter(obp, [ict0, ics0, bl], v0)
                    plsc.store_scatter(obp, [ict1, ics0, bl], v1)

    def issue_write(chunk, ob, sem):
        pltpu.async_copy(
            ob.at[:, :, :, pl.ds(0, _BT)],
            out_hbm.at[pl.ds(chunk * _P, _P), :, wid, :, :],
            sem,
        )

    def wait_write(ob, sem):
        pltpu.make_async_copy(
            ob.at[:, :, :, pl.ds(0, _BT)],
            out_hbm.at[pl.ds(0, _P), :, wid, :, :],
            sem,
        ).wait()

    def body(it, carry):
        c0 = 2 * it

        @pl.when(it > 0)
        def _():
            wait_write(ob0, wsem0)

        compute(c0, ob0)
        issue_write(c0, ob0, wsem0)

        @pl.when(it > 0)
        def _():
            wait_write(ob1, wsem1)

        compute(c0 + 1, ob1)
        issue_write(c0 + 1, ob1, wsem1)
        return carry

    lax.fori_loop(0, n2, body, 0)
    wait_write(ob0, wsem0)
    wait_write(ob1, wsem1)


def kernel(time, table, max_len):
    b, h = time.shape
    v, d = table.shape
    clip = v - 1

    n_pairs = h * h
    assert d % 8 == 0 and n_pairs % (2 * _P) == 0 and _BT % _L == 0
    nbt = b // _BT
    assert nbt == _NW
    nct = d // 8

    time_t = time.T

    mesh = plsc.VectorSubcoreMesh(core_axis_name="c", subcore_axis_name="s")
    out = pl.kernel(
        functools.partial(_body, h, d, clip),
        out_type=jax.ShapeDtypeStruct(
            (n_pairs, nct, nbt, 8, _BT), jnp.float32
        ),
        mesh=mesh,
        scratch_types=[
            pltpu.VMEM((v * d,), jnp.float32),
            pltpu.VMEM((h, _BT), jnp.int32),
            pltpu.VMEM((_P, nct, 8, _BP), jnp.float32),
            pltpu.VMEM((_P, nct, 8, _BP), jnp.float32),
            pltpu.SemaphoreType.DMA,
            pltpu.SemaphoreType.DMA,
        ],
        compiler_params=pltpu.CompilerParams(
            use_tc_tiling_on_sc=False, needs_layout_passes=False
        ),
    )(time_t, table.reshape(v * d))
    phys = out.reshape(h, h, nct, nbt, 8, _BT)
    res = phys.transpose(3, 5, 0, 1, 2, 4)
    return res.reshape(b, h, h, d)


# speedup vs baseline: 1.0597x; 1.0597x over previous
import functools

import jax
import jax.numpy as jnp
from jax import lax
from jax.experimental import pallas as pl
from jax.experimental.pallas import tpu as pltpu
from jax.experimental.pallas import tpu_sc as plsc

_NC = 2
_NS = 16
_NW = _NC * _NS
_L = 16
_BT = 128

_P = 5


def _body(
    h,
    d,
    dp,
    clip,
    time_hbm,
    table_hbm,
    out_hbm,
    table_v,
    t_v,
    ob0,
    ob1,
    wsem0,
    wsem1,
):
    wid = lax.axis_index("s") * _NC + lax.axis_index("c")
    n_pairs = h * h
    n_chunks = n_pairs // _P
    n2 = n_chunks // 2
    n_g = _BT // _L

    pltpu.sync_copy(table_hbm, table_v)
    pltpu.sync_copy(time_hbm.at[:, pl.ds(wid * _BT, _BT)], t_v)

    def compute(chunk, ob):
        p0 = chunk * _P
        for p_loc in range(_P):
            p = p0 + p_loc
            i = p // h
            q = p - i * h

            @plsc.parallel_loop(0, n_g, unroll=1)
            def grp(g):
                gl = g * _L
                ti = t_v[i, pl.ds(gl, _L)]
                tq = t_v[q, pl.ds(gl, _L)]
                rows16 = jnp.minimum(jnp.abs(ti - tq), clip)
                wb = rows16 * dp
                for c in range(d):
                    v = plsc.load_gather(table_v, [wb + c])
                    ob[p_loc, c // 8, pl.ds((c % 8) * _BT + g * _L, _L)] = v

    def issue_write(chunk, ob, sem):
        pltpu.async_copy(
            ob, out_hbm.at[pl.ds(chunk * _P, _P), :, wid, :], sem
        )

    def wait_write(ob, sem):
        pltpu.make_async_copy(
            ob, out_hbm.at[pl.ds(0, _P), :, wid, :], sem
        ).wait()

    def body(it, carry):
        c0 = 2 * it

        @pl.when(it > 0)
        def _():
            wait_write(ob0, wsem0)

        compute(c0, ob0)
        issue_write(c0, ob0, wsem0)

        @pl.when(it > 0)
        def _():
            wait_write(ob1, wsem1)

        compute(c0 + 1, ob1)
        issue_write(c0 + 1, ob1, wsem1)
        return carry

    lax.fori_loop(0, n2, body, 0)
    wait_write(ob0, wsem0)
    wait_write(ob1, wsem1)


def kernel(time, table, max_len):
    b, h = time.shape
    v, d = table.shape
    clip = v - 1

    n_pairs = h * h
    assert b % (_NW * _BT) == 0 or b == _NW * _BT
    assert d % 8 == 0 and n_pairs % (2 * _P) == 0 and _BT % _L == 0
    nbt = b // _BT
    assert nbt == _NW
    nct = d // 8

    dp = d + 1
    table_pad = jnp.concatenate(
        [table, jnp.zeros((v, 1), jnp.float32)], axis=1
    ).reshape(v * dp)

    mesh = plsc.VectorSubcoreMesh(core_axis_name="c", subcore_axis_name="s")
    out = pl.kernel(
        functools.partial(_body, h, d, dp, clip),
        out_type=jax.ShapeDtypeStruct((n_pairs, nct, nbt, 8 * _BT), jnp.float32),
        mesh=mesh,
        scratch_types=[
            pltpu.VMEM((v * dp,), jnp.float32),
            pltpu.VMEM((h, _BT), jnp.int32),
            pltpu.VMEM((_P, nct, 8 * _BT), jnp.float32),
            pltpu.VMEM((_P, nct, 8 * _BT), jnp.float32),
            pltpu.SemaphoreType.DMA,
            pltpu.SemaphoreType.DMA,
        ],
        compiler_params=pltpu.CompilerParams(
            use_tc_tiling_on_sc=False, needs_layout_passes=False
        ),
    )(time.T, table_pad)
    phys = out.reshape(h, h, nct, nbt, 8, _BT)
    res = phys.transpose(3, 5, 0, 1, 2, 4)
    return res.reshape(b, h, h, d)


# speedup vs baseline: 2.8169x; 2.6583x over previous
import functools

import jax
import jax.numpy as jnp
from jax import lax
from jax.experimental import pallas as pl
from jax.experimental.pallas import tpu as pltpu
from jax.experimental.pallas import tpu_sc as plsc

_NC = 2
_NS = 16
_NW = _NC * _NS
_L = 16
_BT = 128


def _body(
    h,
    d,
    dp,
    clip,
    time_hbm,
    table_hbm,
    out_hbm,
    table_v,
    t_v,
    oba0,
    obb0,
    oba1,
    obb1,
    obd,
    sa0,
    sb0,
    sa1,
    sb1,
    sd,
):
    wid = lax.axis_index("s") * _NC + lax.axis_index("c")
    n_g = _BT // _L

    pltpu.sync_copy(table_hbm, table_v)
    pltpu.sync_copy(time_hbm.at[:, pl.ds(wid * _BT, _BT)], t_v)

    zero16 = jnp.zeros((_L,), jnp.int32)

    for c in range(d):
        v = plsc.load_gather(table_v, [zero16 + c])
        for g in range(n_g):
            obd[c // 8, c % 8, pl.ds(g * _L, _L)] = v

    for i in range(h):
        pltpu.async_copy(obd, out_hbm.at[i * h + i, :, wid, :, :], sd)

    def unit(i, q, oba, obb, sema, semb, first):
        @pl.when(jnp.logical_not(first))
        def _():
            pltpu.make_async_copy(
                oba, out_hbm.at[0, :, wid, :, :], sema
            ).wait()
            pltpu.make_async_copy(
                obb, out_hbm.at[0, :, wid, :, :], semb
            ).wait()

        @plsc.parallel_loop(0, n_g, unroll=1)
        def grp(g):
            gl = g * _L
            ti = t_v[i, pl.ds(gl, _L)]
            tq = t_v[q, pl.ds(gl, _L)]
            rows16 = jnp.minimum(jnp.abs(ti - tq), clip)
            wb = rows16 * dp
            for c in range(d):
                v = plsc.load_gather(table_v, [wb + c])
                oba[c // 8, c % 8, pl.ds(gl, _L)] = v
                obb[c // 8, c % 8, pl.ds(gl, _L)] = v

        pltpu.async_copy(oba, out_hbm.at[i * h + q, :, wid, :, :], sema)
        pltpu.async_copy(obb, out_hbm.at[q * h + i, :, wid, :, :], semb)

    def qbody(i, q, u):
        @pl.when(u % 2 == 0)
        def _():
            unit(i, q, oba0, obb0, sa0, sb0, u < 2)

        @pl.when(u % 2 == 1)
        def _():
            unit(i, q, oba1, obb1, sa1, sb1, u < 2)

        return u + 1

    def ibody(i, u):
        return lax.fori_loop(i + 1, h, functools.partial(qbody, i), u)

    u = lax.fori_loop(0, h - 1, ibody, jnp.int32(0))

    @pl.when(u >= 2)
    def _():
        pltpu.make_async_copy(oba0, out_hbm.at[0, :, wid, :, :], sa0).wait()
        pltpu.make_async_copy(obb0, out_hbm.at[0, :, wid, :, :], sb0).wait()

    @pl.when(u >= 1)
    def _():
        pltpu.make_async_copy(oba1, out_hbm.at[0, :, wid, :, :], sa1).wait()
        pltpu.make_async_copy(obb1, out_hbm.at[0, :, wid, :, :], sb1).wait()

    for _i in range(h):
        pltpu.make_async_copy(obd, out_hbm.at[0, :, wid, :, :], sd).wait()


def kernel(time, table, max_len):
    b, h = time.shape
    v, d = table.shape
    clip = v - 1

    assert d % 8 == 0 and _BT % _L == 0
    nbt = b // _BT
    assert nbt == _NW
    nct = d // 8

    dp = d + 1
    table_pad = jnp.concatenate(
        [table, jnp.zeros((v, 1), jnp.float32)], axis=1
    ).reshape(v * dp)

    mesh = plsc.VectorSubcoreMesh(core_axis_name="c", subcore_axis_name="s")
    blk = (nct, 8, _BT)
    out = pl.kernel(
        functools.partial(_body, h, d, dp, clip),
        out_type=jax.ShapeDtypeStruct((h * h, nct, nbt, 8, _BT), jnp.float32),
        mesh=mesh,
        scratch_types=[
            pltpu.VMEM((v * dp,), jnp.float32),
            pltpu.VMEM((h, _BT), jnp.int32),
            pltpu.VMEM(blk, jnp.float32),
            pltpu.VMEM(blk, jnp.float32),
            pltpu.VMEM(blk, jnp.float32),
            pltpu.VMEM(blk, jnp.float32),
            pltpu.VMEM(blk, jnp.float32),
            pltpu.SemaphoreType.DMA,
            pltpu.SemaphoreType.DMA,
            pltpu.SemaphoreType.DMA,
            pltpu.SemaphoreType.DMA,
            pltpu.SemaphoreType.DMA,
        ],
        compiler_params=pltpu.CompilerParams(
            use_tc_tiling_on_sc=False, needs_layout_passes=False
        ),
    )(time.T, table_pad)
    phys = out.reshape(h, h, nct, nbt, 8, _BT)
    res = phys.transpose(3, 5, 0, 1, 2, 4)
    return res.reshape(b, h, h, d)
